# baseline (speedup 1.0000x reference)
import functools

import jax
import jax.numpy as jnp
from jax import lax
from jax.experimental import pallas as pl
from jax.experimental.pallas import tpu as pltpu
from jax.experimental.pallas import tpu_sc as plsc

N = 10000
E = 320000
D = 128
H = 64

NC = 2
NS = 16
NW = NC * NS
LANES = 16

N_PAD = 10240
ROWS_PER_TILE = N_PAD // NS
CHUNK = 128
CH = 80
E_PAD = NW * CH * CHUNK
RD = 8
TOTCH1 = E_PAD // CHUNK
TOTCH2 = E_PAD // 64

A1_K0, A1_K1 = 104, 56
A2_K0, A2_K1 = 288, 32


def _mesh():
    return plsc.VectorSubcoreMesh(
        core_axis_name="c", subcore_axis_name="s", num_cores=NC,
        num_subcores=NS)


_SC_PARAMS = pltpu.CompilerParams(
    needs_layout_passes=False, use_tc_tiling_on_sc=False)


def _rsqrt16(v):
    bits = jax.lax.bitcast_convert_type(v, jnp.int32)
    y = jax.lax.bitcast_convert_type(
        jnp.int32(0x5F3759DF) - jax.lax.shift_right_arithmetic(bits, 1),
        jnp.float32)
    for _ in range(3):
        y = y * (1.5 - 0.5 * v * y * y)
    return y


def _s1_body(ed_hbm, w_hbm, ep1_hbm, ep2_hbm,
             pair_v, w_v, val3_v, val3b_v, deg_v, dis_v, zero_v,
             deg_acc, dis_sp):
    c = lax.axis_index("c")
    s = lax.axis_index("s")
    wid = s * NC + c

    def zloop(i, _):
        zero_v[pl.ds(i * LANES, LANES)] = jnp.zeros((LANES,), jnp.float32)
        return 0
    lax.fori_loop(0, ROWS_PER_TILE // LANES, zloop, 0)
    pltpu.sync_copy(zero_v, deg_acc.at[pl.ds(s * ROWS_PER_TILE, ROWS_PER_TILE)])
    plsc.subcore_barrier()

    def deg_slab(slab):
        pltpu.sync_copy(ed_hbm.at[pl.ds(slab * CH, CH)], pair_v)
        pltpu.sync_copy(w_hbm.at[pl.ds(slab * CH, CH)], w_v)

        def body(j, _):
            pltpu.sync_copy(w_v.at[j], deg_acc.at[pair_v.at[j, 1]], add=True)
            return 0
        lax.fori_loop(0, CH, body, 0)

    deg_slab(2 * s)
    deg_slab(2 * s + 1)
    plsc.subcore_barrier()

    pltpu.sync_copy(deg_acc.at[pl.ds(s * ROWS_PER_TILE, ROWS_PER_TILE)], deg_v)

    def dloop(i, _):
        v = deg_v[pl.ds(i * LANES, LANES)]
        r = _rsqrt16(jnp.maximum(v, 1e-30))
        deg_v[pl.ds(i * LANES, LANES)] = jnp.where(v > 0.0, r, 0.0)
        return 0
    lax.fori_loop(0, ROWS_PER_TILE // LANES, dloop, 0)
    pltpu.sync_copy(deg_v, dis_sp.at[pl.ds(s * ROWS_PER_TILE, ROWS_PER_TILE)])
    plsc.subcore_barrier()

    pltpu.sync_copy(dis_sp, dis_v)
    pltpu.sync_copy(ed_hbm.at[pl.ds(wid * CH, CH)], pair_v)
    pltpu.sync_copy(w_hbm.at[pl.ds(wid * CH, CH)], w_v)

    def srow(j, _):
        def sgrp(g, _):
            sl = pl.ds(g * LANES, LANES)
            si = pair_v[j, 0, sl]
            di = pair_v[j, 1, sl]
            wv = w_v[j, sl]
            d1 = plsc.load_gather(dis_v, [si])
            d2 = plsc.load_gather(dis_v, [di])
            sv = wv * d1 * d2
            val3_v[j, 0, sl] = si
            val3_v[j, 1, sl] = di
            val3_v[j, 2, sl] = plsc.bitcast(sv, jnp.int32)
            return 0
        lax.fori_loop(0, CHUNK // LANES, sgrp, 0)
        return 0
    lax.fori_loop(0, CH, srow, 0)
    pltpu.sync_copy(val3_v, ep1_hbm.at[pl.ds(wid * CH, CH)])

    def brow(j, _):
        for h in range(2):
            for r in range(3):
                for f in range(4):
                    val3b_v[2 * j + h, r, pl.ds(f * LANES, LANES)] = (
                        val3_v[j, r, pl.ds(64 * h + f * LANES, LANES)])
        return 0
    lax.fori_loop(0, CH, brow, 0)
    pltpu.sync_copy(val3b_v, ep2_hbm.at[pl.ds(wid * 2 * CH, 2 * CH)])


def _make_s1():
    return pl.kernel(
        _s1_body,
        out_type=(jax.ShapeDtypeStruct((TOTCH1, 3, CHUNK), jnp.int32),
                  jax.ShapeDtypeStruct((TOTCH2, 3, 64), jnp.int32)),
        mesh=_mesh(),
        compiler_params=_SC_PARAMS,
        scratch_types=dict(
            pair_v=pltpu.VMEM((CH, 2, CHUNK), jnp.int32),
            w_v=pltpu.VMEM((CH, CHUNK), jnp.float32),
            val3_v=pltpu.VMEM((CH, 3, CHUNK), jnp.int32),
            val3b_v=pltpu.VMEM((2 * CH, 3, 64), jnp.int32),
            deg_v=pltpu.VMEM((ROWS_PER_TILE,), jnp.float32),
            dis_v=pltpu.VMEM((N_PAD,), jnp.float32),
            zero_v=pltpu.VMEM((ROWS_PER_TILE,), jnp.float32),
            deg_acc=pltpu.VMEM_SHARED((N_PAD,), jnp.float32),
            dis_sp=pltpu.VMEM_SHARED((N_PAD,), jnp.float32),
        ),
    )


def _agg_body(F, CK, K0, K1, xw0_hbm, xw1_hbm, ep_hbm, part_hbm,
              ring, gbuf, sbuf, acc, rsem, gsem, ssem):
    c = lax.axis_index("c")
    s = lax.axis_index("s")
    FV = F // LANES
    active = (c == 0) if K1 == 0 else (c >= 0)

    @pl.when(active)
    def _():
        def zrow(e, _):
            for f in range(FV):
                sbuf[0, e, pl.ds(f * LANES, LANES)] = jnp.zeros(
                    (LANES,), jnp.float32)
            return 0
        lax.fori_loop(0, CK, zrow, 0)
        for r in range(ROWS_PER_TILE // CK):
            pltpu.sync_copy(
                sbuf.at[0], acc.at[pl.ds(s * ROWS_PER_TILE + r * CK, CK)])
    plsc.subcore_barrier()

    def pipeline(K, base, xw_hbm):
        def slot(j):
            return lax.rem(j, RD) if isinstance(j, jax.Array) else j % RD

        def ring_start(j):
            pltpu.async_copy(ep_hbm.at[base + j], ring.at[slot(j)],
                             rsem.at[slot(j)])

        def ring_wait(j):
            pltpu.make_async_copy(ep_hbm.at[base + j], ring.at[slot(j)],
                                  rsem.at[slot(j)]).wait()

        def gather_start(j, b):
            pltpu.async_copy(xw_hbm.at[ring.at[slot(j), 0]], gbuf.at[b],
                             gsem.at[b])

        def gather_wait(j, b):
            pltpu.make_async_copy(xw_hbm.at[ring.at[slot(j), 0]],
                                  gbuf.at[b], gsem.at[b]).wait()

        def scatter_start(j, b):
            pltpu.async_copy(sbuf.at[b], acc.at[ring.at[slot(j), 1]],
                             ssem.at[b], add=True)

        def scatter_wait(j, b):
            pltpu.make_async_copy(sbuf.at[b], acc.at[ring.at[slot(j), 1]],
                                  ssem.at[b]).wait()

        def scale(j, b):
            def grp(g, _):
                sv16 = plsc.bitcast(
                    ring[slot(j), 2, pl.ds(g * LANES, LANES)], jnp.float32)
                base_e = g * LANES
                for e in range(LANES):
                    sv = jnp.broadcast_to(sv16[e], (LANES,))
                    for f in range(FV):
                        sbuf[b, base_e + e, pl.ds(f * LANES, LANES)] = (
                            gbuf[b, base_e + e, pl.ds(f * LANES, LANES)] * sv)
                return 0
            lax.fori_loop(0, CK // LANES, grp, 0)

        for k in range(6):
            ring_start(k)
        for k in range(2):
            ring_wait(k)
            gather_start(k, k)

        def body(j, _):
            b = lax.rem(j, 2)

            @pl.when(j >= 2)
            def _():
                scatter_wait(j - 2, b)

            @pl.when(j + 6 < K)
            def _():
                ring_start(j + 6)

            @pl.when(j + 2 < K)
            def _():
                ring_wait(j + 2)
            gather_wait(j, b)
            scale(j, b)

            @pl.when(j + 2 < K)
            def _():
                gather_start(j + 2, b)
            scatter_start(j, b)
            return 0
        lax.fori_loop(0, K, body, 0)
        for jj in (K - 2, K - 1):
            scatter_wait(jj, jj % 2)

    @pl.when(c == 0)
    def _():
        pipeline(K0, s * K0, xw0_hbm)

    if K1 > 0:
        @pl.when(c == 1)
        def _():
            pipeline(K1, NS * K0 + s * K1, xw1_hbm)
    plsc.subcore_barrier()

    @pl.when(active)
    def _():
        pltpu.sync_copy(acc.at[pl.ds(s * ROWS_PER_TILE, ROWS_PER_TILE)],
                        part_hbm.at[c, pl.ds(s * ROWS_PER_TILE,
                                             ROWS_PER_TILE)])


def _make_agg(F, CK, K0, K1):
    nparts = 1 if K1 == 0 else 2
    return pl.kernel(
        functools.partial(_agg_body, F, CK, K0, K1),
        out_type=jax.ShapeDtypeStruct((nparts, N_PAD, F), jnp.float32),
        mesh=_mesh(),
        compiler_params=_SC_PARAMS,
        scratch_types=dict(
            ring=pltpu.VMEM((RD, 3, CK), jnp.int32),
            gbuf=pltpu.VMEM((2, CK, F), jnp.float32),
            sbuf=pltpu.VMEM((2, CK, F), jnp.float32),
            acc=pltpu.VMEM_SHARED((N_PAD, F), jnp.float32),
            rsem=pltpu.SemaphoreType.DMA((RD,)),
            gsem=pltpu.SemaphoreType.DMA((2,)),
            ssem=pltpu.SemaphoreType.DMA((2,)),
        ),
    )


def _mm_body(x_ref, w_ref, o_ref, o2_ref):
    r = jnp.dot(x_ref[...], w_ref[...], preferred_element_type=jnp.float32)
    o_ref[...] = r
    o2_ref[...] = r


def _mm(x, w, bm):
    m, k = x.shape
    n = w.shape[1]
    return pl.pallas_call(
        _mm_body,
        grid=(m // bm,),
        in_specs=[pl.BlockSpec((bm, k), lambda i: (i, 0)),
                  pl.BlockSpec((k, n), lambda i: (0, 0))],
        out_specs=[pl.BlockSpec((bm, n), lambda i: (i, 0)),
                   pl.BlockSpec((bm, n), lambda i: (i, 0))],
        out_shape=[jax.ShapeDtypeStruct((m, n), jnp.float32),
                   jax.ShapeDtypeStruct((m, n), jnp.float32)],
    )(x, w)


def _mid_body(p_ref, b_ref, w_ref, o_ref, o2_ref):
    acc = p_ref[0] if p_ref.shape[0] == 1 else p_ref[0] + p_ref[1]
    h = jnp.maximum(acc + b_ref[...], 0.0)
    r = jnp.dot(h, w_ref[...], preferred_element_type=jnp.float32)
    o_ref[...] = r
    o2_ref[...] = r


def _mid(parts, b, w, bm):
    m = parts.shape[1]
    f = parts.shape[2]
    n = w.shape[1]
    return pl.pallas_call(
        _mid_body,
        grid=(m // bm,),
        in_specs=[pl.BlockSpec((parts.shape[0], bm, f), lambda i: (0, i, 0)),
                  pl.BlockSpec((1, f), lambda i: (0, 0)),
                  pl.BlockSpec((f, n), lambda i: (0, 0))],
        out_specs=[pl.BlockSpec((bm, n), lambda i: (i, 0)),
                   pl.BlockSpec((bm, n), lambda i: (i, 0))],
        out_shape=[jax.ShapeDtypeStruct((m, n), jnp.float32),
                   jax.ShapeDtypeStruct((m, n), jnp.float32)],
    )(parts, b.reshape(1, f), w)


def _fin_body(p_ref, b_ref, o_ref):
    acc = p_ref[0] if p_ref.shape[0] == 1 else p_ref[0] + p_ref[1]
    o_ref[...] = jnp.maximum(acc + b_ref[...], 0.0)


def _fin(parts, b, bm):
    m = parts.shape[1]
    f = parts.shape[2]
    return pl.pallas_call(
        _fin_body,
        grid=(m // bm,),
        in_specs=[pl.BlockSpec((parts.shape[0], bm, f), lambda i: (0, i, 0)),
                  pl.BlockSpec((1, f), lambda i: (0, 0))],
        out_specs=pl.BlockSpec((bm, f), lambda i: (i, 0)),
        out_shape=jax.ShapeDtypeStruct((m, f), jnp.float32),
    )(parts, b.reshape(1, f))


def kernel(x, edge_index, edge_weights, W1, b1, W2, b2):
    src = edge_index[0]
    dst = edge_index[1]
    pad = E_PAD - E
    src_r = jnp.concatenate(
        [src, jnp.zeros((pad,), jnp.int32)]).reshape(TOTCH1, CHUNK)
    dst_r = jnp.concatenate(
        [dst, jnp.zeros((pad,), jnp.int32)]).reshape(TOTCH1, CHUNK)
    ed_r = jnp.stack([src_r, dst_r], axis=1)
    w_r = jnp.concatenate(
        [edge_weights, jnp.zeros((pad,), jnp.float32)]).reshape(TOTCH1, CHUNK)

    ep1, ep2 = _make_s1()(ed_r, w_r)

    xw1a, xw1b = _mm(x, W1, 2000)
    p1 = _make_agg(H, CHUNK, A1_K0, A1_K1)(xw1a, xw1b, ep1)
    hw2a, hw2b = _mid(p1, b1, W2, 1280)
    p2 = _make_agg(D, 64, A2_K0, A2_K1)(hw2a, hw2b, ep2)
    out = _fin(p2, b2, 1280)
    return out[:N]


# speedup vs baseline: 1.0153x; 1.0153x over previous
import functools

import jax
import jax.numpy as jnp
from jax import lax
from jax.experimental import pallas as pl
from jax.experimental.pallas import tpu as pltpu
from jax.experimental.pallas import tpu_sc as plsc

N = 10000
E = 320000
D = 128
H = 64

NC = 2
NS = 16
NW = NC * NS
LANES = 16

N_PAD = 10240
ROWS_PER_TILE = N_PAD // NS
CHUNK = 128
CH = 80
E_PAD = NW * CH * CHUNK
RD = 8
TOTCH1 = E_PAD // CHUNK
TOTCH2 = E_PAD // 64

A1_K0, A1_K1 = 95, 65
A2_K0, A2_K1 = 288, 32


def _mesh():
    return plsc.VectorSubcoreMesh(
        core_axis_name="c", subcore_axis_name="s", num_cores=NC,
        num_subcores=NS)


_SC_PARAMS = pltpu.CompilerParams(
    needs_layout_passes=False, use_tc_tiling_on_sc=False)


def _rsqrt16(v):
    bits = jax.lax.bitcast_convert_type(v, jnp.int32)
    y = jax.lax.bitcast_convert_type(
        jnp.int32(0x5F3759DF) - jax.lax.shift_right_arithmetic(bits, 1),
        jnp.float32)
    for _ in range(3):
        y = y * (1.5 - 0.5 * v * y * y)
    return y


def _s1_body(ed_hbm, w_hbm, ep1_hbm, ep2_hbm,
             pair_v, w_v, val3_v, val3b_v, deg_v, dis_v, zero_v,
             deg_acc, dis_sp):
    c = lax.axis_index("c")
    s = lax.axis_index("s")
    wid = s * NC + c

    def zloop(i, _):
        zero_v[pl.ds(i * LANES, LANES)] = jnp.zeros((LANES,), jnp.float32)
        return 0
    lax.fori_loop(0, ROWS_PER_TILE // LANES, zloop, 0)
    pltpu.sync_copy(zero_v, deg_acc.at[pl.ds(s * ROWS_PER_TILE, ROWS_PER_TILE)])
    plsc.subcore_barrier()

    def deg_slab(slab):
        pltpu.sync_copy(ed_hbm.at[pl.ds(slab * CH, CH)], pair_v)
        pltpu.sync_copy(w_hbm.at[pl.ds(slab * CH, CH)], w_v)

        def body(j, _):
            pltpu.sync_copy(w_v.at[j], deg_acc.at[pair_v.at[j, 1]], add=True)
            return 0
        lax.fori_loop(0, CH, body, 0)

    deg_slab(2 * s)
    deg_slab(2 * s + 1)
    plsc.subcore_barrier()

    pltpu.sync_copy(deg_acc.at[pl.ds(s * ROWS_PER_TILE, ROWS_PER_TILE)], deg_v)

    def dloop(i, _):
        v = deg_v[pl.ds(i * LANES, LANES)]
        r = _rsqrt16(jnp.maximum(v, 1e-30))
        deg_v[pl.ds(i * LANES, LANES)] = jnp.where(v > 0.0, r, 0.0)
        return 0
    lax.fori_loop(0, ROWS_PER_TILE // LANES, dloop, 0)
    pltpu.sync_copy(deg_v, dis_sp.at[pl.ds(s * ROWS_PER_TILE, ROWS_PER_TILE)])
    plsc.subcore_barrier()

    pltpu.sync_copy(dis_sp, dis_v)
    pltpu.sync_copy(ed_hbm.at[pl.ds(wid * CH, CH)], pair_v)
    pltpu.sync_copy(w_hbm.at[pl.ds(wid * CH, CH)], w_v)

    def srow(j, _):
        def sgrp(g, _):
            sl = pl.ds(g * LANES, LANES)
            si = pair_v[j, 0, sl]
            di = pair_v[j, 1, sl]
            wv = w_v[j, sl]
            d1 = plsc.load_gather(dis_v, [si])
            d2 = plsc.load_gather(dis_v, [di])
            sv = wv * d1 * d2
            val3_v[j, 0, sl] = si
            val3_v[j, 1, sl] = di
            val3_v[j, 2, sl] = plsc.bitcast(sv, jnp.int32)
            return 0
        lax.fori_loop(0, CHUNK // LANES, sgrp, 0)
        return 0
    lax.fori_loop(0, CH, srow, 0)
    pltpu.sync_copy(val3_v, ep1_hbm.at[pl.ds(wid * CH, CH)])

    def brow(j, _):
        for h in range(2):
            for r in range(3):
                for f in range(4):
                    val3b_v[2 * j + h, r, pl.ds(f * LANES, LANES)] = (
                        val3_v[j, r, pl.ds(64 * h + f * LANES, LANES)])
        return 0
    lax.fori_loop(0, CH, brow, 0)
    pltpu.sync_copy(val3b_v, ep2_hbm.at[pl.ds(wid * 2 * CH, 2 * CH)])


def _make_s1():
    return pl.kernel(
        _s1_body,
        out_type=(jax.ShapeDtypeStruct((TOTCH1, 3, CHUNK), jnp.int32),
                  jax.ShapeDtypeStruct((TOTCH2, 3, 64), jnp.int32)),
        mesh=_mesh(),
        compiler_params=_SC_PARAMS,
        scratch_types=dict(
            pair_v=pltpu.VMEM((CH, 2, CHUNK), jnp.int32),
            w_v=pltpu.VMEM((CH, CHUNK), jnp.float32),
            val3_v=pltpu.VMEM((CH, 3, CHUNK), jnp.int32),
            val3b_v=pltpu.VMEM((2 * CH, 3, 64), jnp.int32),
            deg_v=pltpu.VMEM((ROWS_PER_TILE,), jnp.float32),
            dis_v=pltpu.VMEM((N_PAD,), jnp.float32),
            zero_v=pltpu.VMEM((ROWS_PER_TILE,), jnp.float32),
            deg_acc=pltpu.VMEM_SHARED((N_PAD,), jnp.float32),
            dis_sp=pltpu.VMEM_SHARED((N_PAD,), jnp.float32),
        ),
    )


def _agg_body(F, CK, K0, K1, xw0_hbm, xw1_hbm, ep_hbm, part_hbm,
              ring, gbuf, sbuf, acc, rsem, gsem, ssem):
    c = lax.axis_index("c")
    s = lax.axis_index("s")
    FV = F // LANES
    active = (c == 0) if K1 == 0 else (c >= 0)

    @pl.when(active)
    def _():
        def zrow(e, _):
            for f in range(FV):
                sbuf[0, e, pl.ds(f * LANES, LANES)] = jnp.zeros(
                    (LANES,), jnp.float32)
            return 0
        lax.fori_loop(0, CK, zrow, 0)
        for r in range(ROWS_PER_TILE // CK):
            pltpu.sync_copy(
                sbuf.at[0], acc.at[pl.ds(s * ROWS_PER_TILE + r * CK, CK)])
    plsc.subcore_barrier()

    def pipeline(K, base, xw_hbm):
        def slot(j):
            return lax.rem(j, RD) if isinstance(j, jax.Array) else j % RD

        def ring_start(j):
            pltpu.async_copy(ep_hbm.at[base + j], ring.at[slot(j)],
                             rsem.at[slot(j)])

        def ring_wait(j):
            pltpu.make_async_copy(ep_hbm.at[base + j], ring.at[slot(j)],
                                  rsem.at[slot(j)]).wait()

        def gather_start(j, b):
            pltpu.async_copy(xw_hbm.at[ring.at[slot(j), 0]], gbuf.at[b],
                             gsem.at[b])

        def gather_wait(j, b):
            pltpu.make_async_copy(xw_hbm.at[ring.at[slot(j), 0]],
                                  gbuf.at[b], gsem.at[b]).wait()

        def scatter_start(j, b):
            pltpu.async_copy(sbuf.at[b], acc.at[ring.at[slot(j), 1]],
                             ssem.at[b], add=True)

        def scatter_wait(j, b):
            pltpu.make_async_copy(sbuf.at[b], acc.at[ring.at[slot(j), 1]],
                                  ssem.at[b]).wait()

        def scale(j, b):
            def grp(g, _):
                sv16 = plsc.bitcast(
                    ring[slot(j), 2, pl.ds(g * LANES, LANES)], jnp.float32)
                base_e = g * LANES
                for e in range(LANES):
                    sv = jnp.broadcast_to(sv16[e], (LANES,))
                    for f in range(FV):
                        sbuf[b, base_e + e, pl.ds(f * LANES, LANES)] = (
                            gbuf[b, base_e + e, pl.ds(f * LANES, LANES)] * sv)
                return 0
            lax.fori_loop(0, CK // LANES, grp, 0)

        for k in range(6):
            ring_start(k)
        for k in range(2):
            ring_wait(k)
            gather_start(k, k)

        def body(j, _):
            b = lax.rem(j, 2)

            @pl.when(j >= 2)
            def _():
                scatter_wait(j - 2, b)

            @pl.when(j + 6 < K)
            def _():
                ring_start(j + 6)

            @pl.when(j + 2 < K)
            def _():
                ring_wait(j + 2)
            gather_wait(j, b)
            scale(j, b)

            @pl.when(j + 2 < K)
            def _():
                gather_start(j + 2, b)
            scatter_start(j, b)
            return 0
        lax.fori_loop(0, K, body, 0)
        for jj in (K - 2, K - 1):
            scatter_wait(jj, jj % 2)

    @pl.when(c == 0)
    def _():
        pipeline(K0, s * K0, xw0_hbm)

    if K1 > 0:
        @pl.when(c == 1)
        def _():
            pipeline(K1, NS * K0 + s * K1, xw1_hbm)
    plsc.subcore_barrier()

    @pl.when(active)
    def _():
        pltpu.sync_copy(acc.at[pl.ds(s * ROWS_PER_TILE, ROWS_PER_TILE)],
                        part_hbm.at[c, pl.ds(s * ROWS_PER_TILE,
                                             ROWS_PER_TILE)])


def _make_agg(F, CK, K0, K1):
    nparts = 1 if K1 == 0 else 2
    return pl.kernel(
        functools.partial(_agg_body, F, CK, K0, K1),
        out_type=jax.ShapeDtypeStruct((nparts, N_PAD, F), jnp.float32),
        mesh=_mesh(),
        compiler_params=_SC_PARAMS,
        scratch_types=dict(
            ring=pltpu.VMEM((RD, 3, CK), jnp.int32),
            gbuf=pltpu.VMEM((2, CK, F), jnp.float32),
            sbuf=pltpu.VMEM((2, CK, F), jnp.float32),
            acc=pltpu.VMEM_SHARED((N_PAD, F), jnp.float32),
            rsem=pltpu.SemaphoreType.DMA((RD,)),
            gsem=pltpu.SemaphoreType.DMA((2,)),
            ssem=pltpu.SemaphoreType.DMA((2,)),
        ),
    )


def _mm_body(x_ref, w_ref, o_ref, o2_ref):
    r = jnp.dot(x_ref[...], w_ref[...], preferred_element_type=jnp.float32)
    o_ref[...] = r
    o2_ref[...] = r


def _mm(x, w, bm):
    m, k = x.shape
    n = w.shape[1]
    return pl.pallas_call(
        _mm_body,
        grid=(m // bm,),
        in_specs=[pl.BlockSpec((bm, k), lambda i: (i, 0)),
                  pl.BlockSpec((k, n), lambda i: (0, 0))],
        out_specs=[pl.BlockSpec((bm, n), lambda i: (i, 0)),
                   pl.BlockSpec((bm, n), lambda i: (i, 0))],
        out_shape=[jax.ShapeDtypeStruct((m, n), jnp.float32),
                   jax.ShapeDtypeStruct((m, n), jnp.float32)],
    )(x, w)


def _mid_body(p_ref, b_ref, w_ref, o_ref, o2_ref):
    acc = p_ref[0] if p_ref.shape[0] == 1 else p_ref[0] + p_ref[1]
    h = jnp.maximum(acc + b_ref[...], 0.0)
    r = jnp.dot(h, w_ref[...], preferred_element_type=jnp.float32)
    o_ref[...] = r
    o2_ref[...] = r


def _mid(parts, b, w, bm):
    m = parts.shape[1]
    f = parts.shape[2]
    n = w.shape[1]
    return pl.pallas_call(
        _mid_body,
        grid=(m // bm,),
        in_specs=[pl.BlockSpec((parts.shape[0], bm, f), lambda i: (0, i, 0)),
                  pl.BlockSpec((1, f), lambda i: (0, 0)),
                  pl.BlockSpec((f, n), lambda i: (0, 0))],
        out_specs=[pl.BlockSpec((bm, n), lambda i: (i, 0)),
                   pl.BlockSpec((bm, n), lambda i: (i, 0))],
        out_shape=[jax.ShapeDtypeStruct((m, n), jnp.float32),
                   jax.ShapeDtypeStruct((m, n), jnp.float32)],
    )(parts, b.reshape(1, f), w)


def _fin_body(p_ref, b_ref, o_ref):
    acc = p_ref[0] if p_ref.shape[0] == 1 else p_ref[0] + p_ref[1]
    o_ref[...] = jnp.maximum(acc + b_ref[...], 0.0)


def _fin(parts, b, bm):
    m = parts.shape[1]
    f = parts.shape[2]
    return pl.pallas_call(
        _fin_body,
        grid=(m // bm,),
        in_specs=[pl.BlockSpec((parts.shape[0], bm, f), lambda i: (0, i, 0)),
                  pl.BlockSpec((1, f), lambda i: (0, 0))],
        out_specs=pl.BlockSpec((bm, f), lambda i: (i, 0)),
        out_shape=jax.ShapeDtypeStruct((m, f), jnp.float32),
    )(parts, b.reshape(1, f))


def kernel(x, edge_index, edge_weights, W1, b1, W2, b2):
    src = edge_index[0]
    dst = edge_index[1]
    pad = E_PAD - E
    src_r = jnp.concatenate(
        [src, jnp.zeros((pad,), jnp.int32)]).reshape(TOTCH1, CHUNK)
    dst_r = jnp.concatenate(
        [dst, jnp.zeros((pad,), jnp.int32)]).reshape(TOTCH1, CHUNK)
    ed_r = jnp.stack([src_r, dst_r], axis=1)
    w_r = jnp.concatenate(
        [edge_weights, jnp.zeros((pad,), jnp.float32)]).reshape(TOTCH1, CHUNK)

    ep1, ep2 = _make_s1()(ed_r, w_r)

    xw1a, xw1b = _mm(x, W1, 2000)
    p1 = _make_agg(H, CHUNK, A1_K0, A1_K1)(xw1a, xw1b, ep1)
    hw2a, hw2b = _mid(p1, b1, W2, 1280)
    p2 = _make_agg(D, 64, A2_K0, A2_K1)(hw2a, hw2b, ep2)
    out = _fin(p2, b2, 1280)
    return out[:N]
